# baseline (speedup 1.0000x reference)
import functools

import jax
import jax.numpy as jnp
from jax import lax
from jax.experimental import pallas as pl
from jax.experimental.pallas import tpu as pltpu
from jax.experimental.pallas import tpu_sc as plsc

B, S, C, T = 16, 4, 2, 131072
P = B * S
NW = 32
CHB = 32768
NCH = T // CHB
NB = 3


def _flip_body(y_hbm, left_hbm, out_hbm, left_v, *rest):
    bufs = list(rest[:NB])
    rsems = list(rest[NB:2 * NB])
    wsems = list(rest[2 * NB:3 * NB])
    fsem = rest[3 * NB]

    cid = lax.axis_index("c")
    sid = lax.axis_index("s")
    w = cid * 16 + sid
    pair0 = 2 * w
    b = pair0 // S
    s0 = pair0 % S
    s1 = s0 + 1

    rows = [(s0, 0), (s0, 1), (s1, 0), (s1, 1)]
    xfers = [(s, sc, j * CHB) for (s, sc) in rows for j in range(NCH)]
    n = len(xfers)

    fdesc = pltpu.async_copy(left_hbm, left_v.at[pl.ds(0, P)], fsem)
    rdesc = [None] * NB
    wdesc = [None] * NB
    for t in range(NB):
        s, sc, col = xfers[t]
        rdesc[t] = pltpu.async_copy(
            y_hbm.at[b, s, sc, pl.ds(col, CHB)], bufs[t], rsems[t]
        )
    fdesc.wait()
    lv = left_v[pl.ds(pair0, 16)]
    l0 = lv[0]
    l1 = lv[1]
    dch = [l0, 1 - l0, l1, 1 - l1]

    for t in range(n):
        bb = t % NB
        if t >= NB:
            wdesc[bb].wait()
            s, sc, col = xfers[t]
            rdesc[bb] = pltpu.async_copy(
                y_hbm.at[b, s, sc, pl.ds(col, CHB)], bufs[bb], rsems[bb]
            )
        rdesc[bb].wait()
        s, sc, col = xfers[t]
        wdesc[bb] = pltpu.async_copy(
            bufs[bb], out_hbm.at[b, s, dch[t // NCH], pl.ds(col, CHB)],
            wsems[bb],
        )
    for t in range(n - NB, n):
        wdesc[t % NB].wait()


@jax.jit
def _flip(y, lf):
    mesh = plsc.VectorSubcoreMesh(core_axis_name="c", subcore_axis_name="s")
    return pl.kernel(
        _flip_body,
        out_type=jax.ShapeDtypeStruct((B, S, C, T), jnp.float32),
        mesh=mesh,
        scratch_types=[
            pltpu.VMEM((P + 16,), jnp.int32),
            *[pltpu.VMEM((CHB,), jnp.float32) for _ in range(NB)],
            *[pltpu.SemaphoreType.DMA for _ in range(2 * NB + 1)],
        ],
    )(y, lf)


def kernel(y, left):
    lf = left.reshape(P).astype(jnp.int32)
    return _flip(y, lf)


# speedup vs baseline: 1.0012x; 1.0012x over previous
import functools

import jax
import jax.numpy as jnp
from jax import lax
from jax.experimental import pallas as pl
from jax.experimental.pallas import tpu as pltpu
from jax.experimental.pallas import tpu_sc as plsc

B, S, C, T = 16, 4, 2, 131072
P = B * S
NW = 32
CHB = 32768
NCH = T // CHB
NB = 3


def _flip_body(y_hbm, left_hbm, out_hbm, left_v, *rest):
    bufs = list(rest[:NB])
    rsems = list(rest[NB:2 * NB])
    wsems = list(rest[2 * NB:3 * NB])
    fsem = rest[3 * NB]

    cid = lax.axis_index("c")
    sid = lax.axis_index("s")
    w = cid * 16 + sid
    pair0 = 2 * w
    b = pair0 // S
    s0 = pair0 % S
    s1 = s0 + 1

    rows = [(s0, 0), (s0, 1), (s1, 0), (s1, 1)]
    xfers = [(s, sc, j * CHB) for (s, sc) in rows for j in range(NCH)]
    n = len(xfers)

    def read(i):
        s, sc, col = xfers[i]
        return pltpu.async_copy(
            y_hbm.at[b, s, sc, pl.ds(col, CHB)], bufs[i % NB], rsems[i % NB]
        )

    fdesc = pltpu.async_copy(left_hbm, left_v.at[pl.ds(0, P)], fsem)
    rdesc = [None] * NB
    wdesc = [None] * NB
    for t in range(NB - 1):
        rdesc[t] = read(t)
    fdesc.wait()
    lv = left_v[pl.ds(pair0, 16)]
    l0 = lv[0]
    l1 = lv[1]
    dch = [l0, 1 - l0, l1, 1 - l1]

    for k in range(n):
        rdesc[k % NB].wait()
        s, sc, col = xfers[k]
        wdesc[k % NB] = pltpu.async_copy(
            bufs[k % NB], out_hbm.at[b, s, dch[k // NCH], pl.ds(col, CHB)],
            wsems[k % NB],
        )
        nxt = k + NB - 1
        if nxt < n:
            if k >= 1:
                wdesc[(k - 1) % NB].wait()
            rdesc[nxt % NB] = read(nxt)
    for j in range(n - NB, n):
        wdesc[j % NB].wait()


@jax.jit
def _flip(y, lf):
    mesh = plsc.VectorSubcoreMesh(core_axis_name="c", subcore_axis_name="s")
    return pl.kernel(
        _flip_body,
        out_type=jax.ShapeDtypeStruct((B, S, C, T), jnp.float32),
        mesh=mesh,
        scratch_types=[
            pltpu.VMEM((P + 16,), jnp.int32),
            *[pltpu.VMEM((CHB,), jnp.float32) for _ in range(NB)],
            *[pltpu.SemaphoreType.DMA for _ in range(2 * NB + 1)],
        ],
    )(y, lf)


def kernel(y, left):
    lf = left.reshape(P).astype(jnp.int32)
    return _flip(y, lf)
